# baseline (speedup 1.0000x reference)
import functools

import jax
import jax.numpy as jnp
from jax import lax
from jax.experimental import pallas as pl
from jax.experimental.pallas import tpu as pltpu
from jax.experimental.pallas import tpu_sc as plsc

L = 16
NW = 32
NB = 8
N = 2048
CHUNK = (NB * N) // NW
WPB = N // CHUNK
PV = CHUNK // L
PC = 512
JC = 512


def _bf(x):
    return x.astype(jnp.bfloat16).astype(jnp.float32)


SEG = 256


def _seg_scan(d, base, winners):
    jj = lax.broadcasted_iota(jnp.int32, (PC, SEG), 1)
    for c in range(d.shape[1] // SEG):
        dc = jnp.maximum(d[:, c * SEG:(c + 1) * SEG], 0.0)
        bits = lax.bitcast_convert_type(dc, jnp.int32)
        k = jnp.min((bits & jnp.int32(-SEG)) | jj, axis=1)
        winners.append((k & jnp.int32(-SEG),
                        (k & jnp.int32(SEG - 1)) + (base + c * SEG)))


def _argmin_body(predT_ref, pred_ref, tgt_ref, idx_ref):
    aT = predT_ref[0]
    ap = pred_ref[0]
    t = tgt_ref[0]
    ra = (ap[0] * ap[0] + ap[1] * ap[1]) + ap[2] * ap[2]
    rac = ra[:, None]
    H = N // 2
    winners = []

    t0 = t[:, :H]
    rb0 = (t0[0] * t0[0] + t0[1] * t0[1]) + t0[2] * t0[2]
    m2a = lax.dot_general(
        aT.astype(jnp.bfloat16),
        (t0 * 2.0).astype(jnp.bfloat16),
        (((1,), (0,)), ((), ())), preferred_element_type=jnp.float32)
    _seg_scan((rac - m2a) + rb0[None, :], 0, winners)

    pxb = _bf(ap[0])[:, None]
    pyb = _bf(ap[1])[:, None]
    pzb = _bf(ap[2])[:, None]
    t1 = t[:, H:]
    rb1 = (t1[0] * t1[0] + t1[1] * t1[1]) + t1[2] * t1[2]
    tx2 = (_bf(t1[0]) * 2.0)[None, :]
    ty2 = (_bf(t1[1]) * 2.0)[None, :]
    tz2 = (_bf(t1[2]) * 2.0)[None, :]
    m2b = (pxb * tx2 + pzb * tz2) + pyb * ty2
    _seg_scan((rac - m2b) + rb1[None, :], H, winners)

    bd, bj = winners[0]
    for kd, kj in winners[1:]:
        take = kd < bd
        bd = jnp.where(take, kd, bd)
        bj = jnp.where(take, kj, bj)
    idx_ref[0, 0] = bj


_tc_argmin = pl.pallas_call(
    _argmin_body,
    grid=(NB, N // PC),
    in_specs=[
        pl.BlockSpec((1, PC, 3), lambda b, i: (b, i, 0)),
        pl.BlockSpec((1, 3, PC), lambda b, i: (b, 0, i)),
        pl.BlockSpec((1, 3, N), lambda b, i: (b, 0, 0)),
    ],
    out_specs=pl.BlockSpec((1, 1, PC), lambda b, i: (b * (N // PC) + i, 0, 0)),
    out_shape=jax.ShapeDtypeStruct((NB * (N // PC), 1, PC), jnp.int32),
)


def _gather_body(pred_hbm, target_hbm, idx_hbm, out_hbm,
                 px, py, pz, tx, ty, tz, ib, ob):
    w = lax.axis_index("s") * 2 + lax.axis_index("c")
    b = w // WPB
    off = (w % WPB) * CHUNK

    pltpu.sync_copy(pred_hbm.at[pl.ds((b * 3 + 0) * N + off, CHUNK)], px)
    pltpu.sync_copy(pred_hbm.at[pl.ds((b * 3 + 1) * N + off, CHUNK)], py)
    pltpu.sync_copy(pred_hbm.at[pl.ds((b * 3 + 2) * N + off, CHUNK)], pz)
    pltpu.sync_copy(target_hbm.at[pl.ds((b * 3 + 0) * N, N)], tx)
    pltpu.sync_copy(target_hbm.at[pl.ds((b * 3 + 1) * N, N)], ty)
    pltpu.sync_copy(target_hbm.at[pl.ds((b * 3 + 2) * N, N)], tz)
    pltpu.sync_copy(idx_hbm.at[pl.ds(w * CHUNK, CHUNK)], ib)

    def pred_loop(ip, carry):
        s = pl.ds(ip * L, L)
        bj = ib[s]
        gx = plsc.load_gather(tx, [bj])
        gy = plsc.load_gather(ty, [bj])
        gz = plsc.load_gather(tz, [bj])
        dx = px[s] - gx
        dy = py[s] - gy
        dz = pz[s] - gz
        d2 = (dx * dx + dy * dy) + dz * dz

        yi = lax.bitcast_convert_type(d2, jnp.int32)
        y = lax.bitcast_convert_type(
            (yi >> 1) + jnp.int32(0x1FBD1DF5), jnp.float32)
        y = 0.5 * (y + d2 / y)
        y = 0.5 * (y + d2 / y)
        y = 0.5 * (y + d2 / y)
        ob[s] = y
        return carry

    lax.fori_loop(0, PV, pred_loop, 0, unroll=1)
    pltpu.sync_copy(ob, out_hbm.at[pl.ds(w * CHUNK, CHUNK)])


_sc_gather = pl.kernel(
    _gather_body,
    out_type=jax.ShapeDtypeStruct((NB * N,), jnp.float32),
    mesh=plsc.VectorSubcoreMesh(core_axis_name="c", subcore_axis_name="s"),
    compiler_params=pltpu.CompilerParams(needs_layout_passes=False),
    scratch_types=[
        pltpu.VMEM((CHUNK,), jnp.float32),
        pltpu.VMEM((CHUNK,), jnp.float32),
        pltpu.VMEM((CHUNK,), jnp.float32),
        pltpu.VMEM((N,), jnp.float32),
        pltpu.VMEM((N,), jnp.float32),
        pltpu.VMEM((N,), jnp.float32),
        pltpu.VMEM((CHUNK,), jnp.int32),
        pltpu.VMEM((CHUNK,), jnp.float32),
    ],
)


@jax.jit
def kernel(pred, target):
    B, T, d, n = pred.shape
    pred2 = pred.reshape(NB, 3, N)
    target2 = target.reshape(NB, 3, N)
    predT = jnp.transpose(pred2, (0, 2, 1))
    idx = _tc_argmin(predT, pred2, target2).reshape(NB * N)
    out = _sc_gather(pred2.reshape(-1), target2.reshape(-1), idx)
    return out.reshape(NB, N)


# speedup vs baseline: 1.6998x; 1.6998x over previous
import functools

import jax
import jax.numpy as jnp
from jax import lax
from jax.experimental import pallas as pl
from jax.experimental.pallas import tpu as pltpu
from jax.experimental.pallas import tpu_sc as plsc

L = 16
NW = 32
NB = 8
N = 2048
CHUNK = (NB * N) // NW
WPB = N // CHUNK
PV = CHUNK // L
PC = 512
JC = 512


def _bf(x):
    return x.astype(jnp.bfloat16).astype(jnp.float32)


def _argmin_body(predT_ref, pred_ref, tgt_ref, idx_ref):
    aT = predT_ref[0]
    ap = pred_ref[0]
    t = tgt_ref[0]
    ra = (ap[0] * ap[0] + ap[1] * ap[1]) + ap[2] * ap[2]
    rb = (t[0] * t[0] + t[1] * t[1]) + t[2] * t[2]
    m2 = lax.dot_general(
        aT.astype(jnp.bfloat16),
        (t * 2.0).astype(jnp.bfloat16),
        (((1,), (0,)), ((), ())), preferred_element_type=jnp.float32)
    d = (ra[:, None] - m2) + rb[None, :]
    idx_ref[0, 0] = jnp.argmin(d, axis=1).astype(jnp.int32)


_tc_argmin = pl.pallas_call(
    _argmin_body,
    grid=(NB, N // PC),
    in_specs=[
        pl.BlockSpec((1, PC, 3), lambda b, i: (b, i, 0)),
        pl.BlockSpec((1, 3, PC), lambda b, i: (b, 0, i)),
        pl.BlockSpec((1, 3, N), lambda b, i: (b, 0, 0)),
    ],
    out_specs=pl.BlockSpec((1, 1, PC), lambda b, i: (b * (N // PC) + i, 0, 0)),
    out_shape=jax.ShapeDtypeStruct((NB * (N // PC), 1, PC), jnp.int32),
)


def _gather_body(pred_hbm, target_hbm, idx_hbm, out_hbm,
                 px, py, pz, tx, ty, tz, ib, ob):
    w = lax.axis_index("s") * 2 + lax.axis_index("c")
    b = w // WPB
    off = (w % WPB) * CHUNK

    pltpu.sync_copy(pred_hbm.at[pl.ds((b * 3 + 0) * N + off, CHUNK)], px)
    pltpu.sync_copy(pred_hbm.at[pl.ds((b * 3 + 1) * N + off, CHUNK)], py)
    pltpu.sync_copy(pred_hbm.at[pl.ds((b * 3 + 2) * N + off, CHUNK)], pz)
    pltpu.sync_copy(target_hbm.at[pl.ds((b * 3 + 0) * N, N)], tx)
    pltpu.sync_copy(target_hbm.at[pl.ds((b * 3 + 1) * N, N)], ty)
    pltpu.sync_copy(target_hbm.at[pl.ds((b * 3 + 2) * N, N)], tz)
    pltpu.sync_copy(idx_hbm.at[pl.ds(w * CHUNK, CHUNK)], ib)

    def pred_loop(ip, carry):
        s = pl.ds(ip * L, L)
        bj = ib[s]
        gx = plsc.load_gather(tx, [bj])
        gy = plsc.load_gather(ty, [bj])
        gz = plsc.load_gather(tz, [bj])
        dx = px[s] - gx
        dy = py[s] - gy
        dz = pz[s] - gz
        d2 = (dx * dx + dy * dy) + dz * dz

        yi = lax.bitcast_convert_type(d2, jnp.int32)
        y = lax.bitcast_convert_type(
            (yi >> 1) + jnp.int32(0x1FBD1DF5), jnp.float32)
        y = 0.5 * (y + d2 / y)
        y = 0.5 * (y + d2 / y)
        y = 0.5 * (y + d2 / y)
        ob[s] = y
        return carry

    lax.fori_loop(0, PV, pred_loop, 0, unroll=1)
    pltpu.sync_copy(ob, out_hbm.at[pl.ds(w * CHUNK, CHUNK)])


_sc_gather = pl.kernel(
    _gather_body,
    out_type=jax.ShapeDtypeStruct((NB * N,), jnp.float32),
    mesh=plsc.VectorSubcoreMesh(core_axis_name="c", subcore_axis_name="s"),
    compiler_params=pltpu.CompilerParams(needs_layout_passes=False),
    scratch_types=[
        pltpu.VMEM((CHUNK,), jnp.float32),
        pltpu.VMEM((CHUNK,), jnp.float32),
        pltpu.VMEM((CHUNK,), jnp.float32),
        pltpu.VMEM((N,), jnp.float32),
        pltpu.VMEM((N,), jnp.float32),
        pltpu.VMEM((N,), jnp.float32),
        pltpu.VMEM((CHUNK,), jnp.int32),
        pltpu.VMEM((CHUNK,), jnp.float32),
    ],
)


@jax.jit
def kernel(pred, target):
    B, T, d, n = pred.shape
    pred2 = pred.reshape(NB, 3, N)
    target2 = target.reshape(NB, 3, N)
    predT = jnp.transpose(pred2, (0, 2, 1))
    idx = _tc_argmin(predT, pred2, target2).reshape(NB * N)
    out = _sc_gather(pred2.reshape(-1), target2.reshape(-1), idx)
    return out.reshape(NB, N)


# speedup vs baseline: 1.7632x; 1.0373x over previous
import functools

import jax
import jax.numpy as jnp
from jax import lax
from jax.experimental import pallas as pl
from jax.experimental.pallas import tpu as pltpu
from jax.experimental.pallas import tpu_sc as plsc

L = 16
NW = 32
NB = 8
N = 2048
CHUNK = (NB * N) // NW
WPB = N // CHUNK
PV = CHUNK // L
PC = 512
JC = 512


def _bf(x):
    return x.astype(jnp.bfloat16).astype(jnp.float32)


def _argmin_body(pred_ref, tgt_ref, idx_ref):
    ap = pred_ref[0]
    t = tgt_ref[0]
    ra = (ap[0] * ap[0] + ap[1] * ap[1]) + ap[2] * ap[2]
    rb = (t[0] * t[0] + t[1] * t[1]) + t[2] * t[2]
    m2 = lax.dot_general(
        ap.astype(jnp.bfloat16),
        (t * 2.0).astype(jnp.bfloat16),
        (((0,), (0,)), ((), ())), preferred_element_type=jnp.float32)
    d = (ra[:, None] - m2) + rb[None, :]
    idx_ref[0, 0] = jnp.argmin(d, axis=1).astype(jnp.int32)


_tc_argmin = pl.pallas_call(
    _argmin_body,
    grid=(NB, N // PC),
    in_specs=[
        pl.BlockSpec((1, 3, PC), lambda b, i: (b, 0, i)),
        pl.BlockSpec((1, 3, N), lambda b, i: (b, 0, 0)),
    ],
    out_specs=pl.BlockSpec((1, 1, PC), lambda b, i: (b * (N // PC) + i, 0, 0)),
    out_shape=jax.ShapeDtypeStruct((NB * (N // PC), 1, PC), jnp.int32),
)


def _gather_body(pred_hbm, target_hbm, idx_hbm, out_hbm,
                 px, py, pz, tx, ty, tz, ib, ob):
    w = lax.axis_index("s") * 2 + lax.axis_index("c")
    b = w // WPB
    off = (w % WPB) * CHUNK

    pltpu.sync_copy(pred_hbm.at[pl.ds((b * 3 + 0) * N + off, CHUNK)], px)
    pltpu.sync_copy(pred_hbm.at[pl.ds((b * 3 + 1) * N + off, CHUNK)], py)
    pltpu.sync_copy(pred_hbm.at[pl.ds((b * 3 + 2) * N + off, CHUNK)], pz)
    pltpu.sync_copy(target_hbm.at[pl.ds((b * 3 + 0) * N, N)], tx)
    pltpu.sync_copy(target_hbm.at[pl.ds((b * 3 + 1) * N, N)], ty)
    pltpu.sync_copy(target_hbm.at[pl.ds((b * 3 + 2) * N, N)], tz)
    pltpu.sync_copy(idx_hbm.at[pl.ds(w * CHUNK, CHUNK)], ib)

    def pred_loop(ip, carry):
        s = pl.ds(ip * L, L)
        bj = ib[s]
        gx = plsc.load_gather(tx, [bj])
        gy = plsc.load_gather(ty, [bj])
        gz = plsc.load_gather(tz, [bj])
        dx = px[s] - gx
        dy = py[s] - gy
        dz = pz[s] - gz
        d2 = (dx * dx + dy * dy) + dz * dz

        yi = lax.bitcast_convert_type(d2, jnp.int32)
        y = lax.bitcast_convert_type(
            (yi >> 1) + jnp.int32(0x1FBD1DF5), jnp.float32)
        y = 0.5 * (y + d2 / y)
        y = 0.5 * (y + d2 / y)
        y = 0.5 * (y + d2 / y)
        ob[s] = y
        return carry

    lax.fori_loop(0, PV, pred_loop, 0, unroll=1)
    pltpu.sync_copy(ob, out_hbm.at[pl.ds(w * CHUNK, CHUNK)])


_sc_gather = pl.kernel(
    _gather_body,
    out_type=jax.ShapeDtypeStruct((NB * N,), jnp.float32),
    mesh=plsc.VectorSubcoreMesh(core_axis_name="c", subcore_axis_name="s"),
    compiler_params=pltpu.CompilerParams(needs_layout_passes=False),
    scratch_types=[
        pltpu.VMEM((CHUNK,), jnp.float32),
        pltpu.VMEM((CHUNK,), jnp.float32),
        pltpu.VMEM((CHUNK,), jnp.float32),
        pltpu.VMEM((N,), jnp.float32),
        pltpu.VMEM((N,), jnp.float32),
        pltpu.VMEM((N,), jnp.float32),
        pltpu.VMEM((CHUNK,), jnp.int32),
        pltpu.VMEM((CHUNK,), jnp.float32),
    ],
)


@jax.jit
def kernel(pred, target):
    B, T, d, n = pred.shape
    pred2 = pred.reshape(NB, 3, N)
    target2 = target.reshape(NB, 3, N)
    idx = _tc_argmin(pred2, target2).reshape(NB * N)
    out = _sc_gather(pred2.reshape(-1), target2.reshape(-1), idx)
    return out.reshape(NB, N)


# speedup vs baseline: 1.8374x; 1.0421x over previous
import functools

import jax
import jax.numpy as jnp
from jax import lax
from jax.experimental import pallas as pl
from jax.experimental.pallas import tpu as pltpu
from jax.experimental.pallas import tpu_sc as plsc

L = 16
NW = 32
NB = 8
N = 2048
CHUNK = (NB * N) // NW
WPB = N // CHUNK
PV = CHUNK // L
PC = 1024
JC = 512


def _bf(x):
    return x.astype(jnp.bfloat16).astype(jnp.float32)


def _argmin_body(pred_ref, tgt_ref, idx_ref):
    ap = pred_ref[0]
    t = tgt_ref[0]
    ra = (ap[0] * ap[0] + ap[1] * ap[1]) + ap[2] * ap[2]
    rb = (t[0] * t[0] + t[1] * t[1]) + t[2] * t[2]
    m2 = lax.dot_general(
        ap.astype(jnp.bfloat16),
        (t * 2.0).astype(jnp.bfloat16),
        (((0,), (0,)), ((), ())), preferred_element_type=jnp.float32)
    d = (ra[:, None] - m2) + rb[None, :]
    idx_ref[0, 0] = jnp.argmin(d, axis=1).astype(jnp.int32)


_tc_argmin = pl.pallas_call(
    _argmin_body,
    grid=(NB, N // PC),
    in_specs=[
        pl.BlockSpec((1, 3, PC), lambda b, i: (b, 0, i)),
        pl.BlockSpec((1, 3, N), lambda b, i: (b, 0, 0)),
    ],
    out_specs=pl.BlockSpec((1, 1, PC), lambda b, i: (b * (N // PC) + i, 0, 0)),
    out_shape=jax.ShapeDtypeStruct((NB * (N // PC), 1, PC), jnp.int32),
)


def _gather_body(pred_hbm, target_hbm, idx_hbm, out_hbm,
                 px, py, pz, tx, ty, tz, ib, ob):
    w = lax.axis_index("s") * 2 + lax.axis_index("c")
    b = w // WPB
    off = (w % WPB) * CHUNK

    pltpu.sync_copy(pred_hbm.at[pl.ds((b * 3 + 0) * N + off, CHUNK)], px)
    pltpu.sync_copy(pred_hbm.at[pl.ds((b * 3 + 1) * N + off, CHUNK)], py)
    pltpu.sync_copy(pred_hbm.at[pl.ds((b * 3 + 2) * N + off, CHUNK)], pz)
    pltpu.sync_copy(target_hbm.at[pl.ds((b * 3 + 0) * N, N)], tx)
    pltpu.sync_copy(target_hbm.at[pl.ds((b * 3 + 1) * N, N)], ty)
    pltpu.sync_copy(target_hbm.at[pl.ds((b * 3 + 2) * N, N)], tz)
    pltpu.sync_copy(idx_hbm.at[pl.ds(w * CHUNK, CHUNK)], ib)

    def pred_loop(ip, carry):
        s = pl.ds(ip * L, L)
        bj = ib[s]
        gx = plsc.load_gather(tx, [bj])
        gy = plsc.load_gather(ty, [bj])
        gz = plsc.load_gather(tz, [bj])
        dx = px[s] - gx
        dy = py[s] - gy
        dz = pz[s] - gz
        d2 = (dx * dx + dy * dy) + dz * dz

        yi = lax.bitcast_convert_type(d2, jnp.int32)
        y = lax.bitcast_convert_type(
            (yi >> 1) + jnp.int32(0x1FBD1DF5), jnp.float32)
        y = 0.5 * (y + d2 / y)
        y = 0.5 * (y + d2 / y)
        y = 0.5 * (y + d2 / y)
        ob[s] = y
        return carry

    lax.fori_loop(0, PV, pred_loop, 0, unroll=1)
    pltpu.sync_copy(ob, out_hbm.at[pl.ds(w * CHUNK, CHUNK)])


_sc_gather = pl.kernel(
    _gather_body,
    out_type=jax.ShapeDtypeStruct((NB * N,), jnp.float32),
    mesh=plsc.VectorSubcoreMesh(core_axis_name="c", subcore_axis_name="s"),
    compiler_params=pltpu.CompilerParams(needs_layout_passes=False),
    scratch_types=[
        pltpu.VMEM((CHUNK,), jnp.float32),
        pltpu.VMEM((CHUNK,), jnp.float32),
        pltpu.VMEM((CHUNK,), jnp.float32),
        pltpu.VMEM((N,), jnp.float32),
        pltpu.VMEM((N,), jnp.float32),
        pltpu.VMEM((N,), jnp.float32),
        pltpu.VMEM((CHUNK,), jnp.int32),
        pltpu.VMEM((CHUNK,), jnp.float32),
    ],
)


@jax.jit
def kernel(pred, target):
    B, T, d, n = pred.shape
    pred2 = pred.reshape(NB, 3, N)
    target2 = target.reshape(NB, 3, N)
    idx = _tc_argmin(pred2, target2).reshape(NB * N)
    out = _sc_gather(pred2.reshape(-1), target2.reshape(-1), idx)
    return out.reshape(NB, N)


# speedup vs baseline: 1.9363x; 1.0538x over previous
import functools

import jax
import jax.numpy as jnp
from jax import lax
from jax.experimental import pallas as pl
from jax.experimental.pallas import tpu as pltpu
from jax.experimental.pallas import tpu_sc as plsc

L = 16
NW = 32
NB = 8
N = 2048
CHUNK = (NB * N) // NW
WPB = N // CHUNK
PV = CHUNK // L
PC = 1024
JC = 512


def _bf(x):
    return x.astype(jnp.bfloat16).astype(jnp.float32)


def _argmin_body(pred_ref, tgt_ref, idx_ref):
    ap = pred_ref[0]
    t = tgt_ref[0]
    rb = (t[0] * t[0] + t[1] * t[1]) + t[2] * t[2]
    m2 = lax.dot_general(
        ap.astype(jnp.bfloat16),
        (t * 2.0).astype(jnp.bfloat16),
        (((0,), (0,)), ((), ())), preferred_element_type=jnp.float32)
    d = rb[None, :] - m2
    idx_ref[0, 0] = jnp.argmin(d, axis=1).astype(jnp.int32)


_tc_argmin = pl.pallas_call(
    _argmin_body,
    grid=(NB, N // PC),
    in_specs=[
        pl.BlockSpec((1, 3, PC), lambda b, i: (b, 0, i)),
        pl.BlockSpec((1, 3, N), lambda b, i: (b, 0, 0)),
    ],
    out_specs=pl.BlockSpec((1, 1, PC), lambda b, i: (b * (N // PC) + i, 0, 0)),
    out_shape=jax.ShapeDtypeStruct((NB * (N // PC), 1, PC), jnp.int32),
)


def _gather_body(pred_hbm, target_hbm, idx_hbm, out_hbm,
                 px, py, pz, tx, ty, tz, ib, ob, dsem):
    w = lax.axis_index("s") * 2 + lax.axis_index("c")
    b = w // WPB
    off = (w % WPB) * CHUNK

    cps = [
        pltpu.async_copy(pred_hbm.at[pl.ds((b * 3 + 0) * N + off, CHUNK)],
                         px, dsem),
        pltpu.async_copy(pred_hbm.at[pl.ds((b * 3 + 1) * N + off, CHUNK)],
                         py, dsem),
        pltpu.async_copy(pred_hbm.at[pl.ds((b * 3 + 2) * N + off, CHUNK)],
                         pz, dsem),
        pltpu.async_copy(target_hbm.at[pl.ds((b * 3 + 0) * N, N)], tx, dsem),
        pltpu.async_copy(target_hbm.at[pl.ds((b * 3 + 1) * N, N)], ty, dsem),
        pltpu.async_copy(target_hbm.at[pl.ds((b * 3 + 2) * N, N)], tz, dsem),
        pltpu.async_copy(idx_hbm.at[pl.ds(w * CHUNK, CHUNK)], ib, dsem),
    ]
    for c in cps:
        c.wait()

    def pred_loop(ip, carry):
        s = pl.ds(ip * L, L)
        bj = ib[s]
        gx = plsc.load_gather(tx, [bj])
        gy = plsc.load_gather(ty, [bj])
        gz = plsc.load_gather(tz, [bj])
        dx = px[s] - gx
        dy = py[s] - gy
        dz = pz[s] - gz
        d2 = (dx * dx + dy * dy) + dz * dz

        yi = lax.bitcast_convert_type(d2, jnp.int32)
        y = lax.bitcast_convert_type(
            (yi >> 1) + jnp.int32(0x1FBD1DF5), jnp.float32)
        y = 0.5 * (y + d2 / y)
        y = 0.5 * (y + d2 / y)
        y = 0.5 * (y + d2 / y)
        ob[s] = y
        return carry

    lax.fori_loop(0, PV, pred_loop, 0, unroll=1)
    pltpu.sync_copy(ob, out_hbm.at[pl.ds(w * CHUNK, CHUNK)])


_sc_gather = pl.kernel(
    _gather_body,
    out_type=jax.ShapeDtypeStruct((NB * N,), jnp.float32),
    mesh=plsc.VectorSubcoreMesh(core_axis_name="c", subcore_axis_name="s"),
    compiler_params=pltpu.CompilerParams(needs_layout_passes=False),
    scratch_types=[
        pltpu.VMEM((CHUNK,), jnp.float32),
        pltpu.VMEM((CHUNK,), jnp.float32),
        pltpu.VMEM((CHUNK,), jnp.float32),
        pltpu.VMEM((N,), jnp.float32),
        pltpu.VMEM((N,), jnp.float32),
        pltpu.VMEM((N,), jnp.float32),
        pltpu.VMEM((CHUNK,), jnp.int32),
        pltpu.VMEM((CHUNK,), jnp.float32),
        pltpu.SemaphoreType.DMA,
    ],
)


@jax.jit
def kernel(pred, target):
    B, T, d, n = pred.shape
    pred2 = pred.reshape(NB, 3, N)
    target2 = target.reshape(NB, 3, N)
    idx = _tc_argmin(pred2, target2).reshape(NB * N)
    out = _sc_gather(pred2.reshape(-1), target2.reshape(-1), idx)
    return out.reshape(NB, N)


# speedup vs baseline: 1.9377x; 1.0007x over previous
import jax
import jax.numpy as jnp
from jax import lax
from jax.experimental import pallas as pl
from jax.experimental.pallas import tpu as pltpu
from jax.experimental.pallas import tpu_sc as plsc

L = 16
NW = 32
NB = 8
N = 2048
CHUNK = (NB * N) // NW
WPB = N // CHUNK
PV = CHUNK // L
PC = 1024


def _argmin_body(pred_ref, tgt_ref, idx_ref):
    ap = pred_ref[0]
    t = tgt_ref[0]
    rb = (t[0] * t[0] + t[1] * t[1]) + t[2] * t[2]
    m2 = lax.dot_general(
        ap.astype(jnp.bfloat16),
        (t * 2.0).astype(jnp.bfloat16),
        (((0,), (0,)), ((), ())), preferred_element_type=jnp.float32)
    d = rb[None, :] - m2
    idx_ref[0, 0] = jnp.argmin(d, axis=1).astype(jnp.int32)


_tc_argmin = pl.pallas_call(
    _argmin_body,
    grid=(NB, N // PC),
    in_specs=[
        pl.BlockSpec((1, 3, PC), lambda b, i: (b, 0, i)),
        pl.BlockSpec((1, 3, N), lambda b, i: (b, 0, 0)),
    ],
    out_specs=pl.BlockSpec((1, 1, PC), lambda b, i: (b * (N // PC) + i, 0, 0)),
    out_shape=jax.ShapeDtypeStruct((NB * (N // PC), 1, PC), jnp.int32),
)


def _gather_body(pred_hbm, target_hbm, idx_hbm, out_hbm,
                 px, py, pz, tx, ty, tz, ib, ob, dsem):
    w = lax.axis_index("s") * 2 + lax.axis_index("c")
    b = w // WPB
    off = (w % WPB) * CHUNK

    cps = [
        pltpu.async_copy(pred_hbm.at[pl.ds((b * 3 + 0) * N + off, CHUNK)],
                         px, dsem),
        pltpu.async_copy(pred_hbm.at[pl.ds((b * 3 + 1) * N + off, CHUNK)],
                         py, dsem),
        pltpu.async_copy(pred_hbm.at[pl.ds((b * 3 + 2) * N + off, CHUNK)],
                         pz, dsem),
        pltpu.async_copy(target_hbm.at[pl.ds((b * 3 + 0) * N, N)], tx, dsem),
        pltpu.async_copy(target_hbm.at[pl.ds((b * 3 + 1) * N, N)], ty, dsem),
        pltpu.async_copy(target_hbm.at[pl.ds((b * 3 + 2) * N, N)], tz, dsem),
        pltpu.async_copy(idx_hbm.at[pl.ds(w * CHUNK, CHUNK)], ib, dsem),
    ]
    for c in cps:
        c.wait()

    def pred_loop(ip, carry):
        s = pl.ds(ip * L, L)
        bj = ib[s]
        gx = plsc.load_gather(tx, [bj])
        gy = plsc.load_gather(ty, [bj])
        gz = plsc.load_gather(tz, [bj])
        dx = px[s] - gx
        dy = py[s] - gy
        dz = pz[s] - gz
        d2 = (dx * dx + dy * dy) + dz * dz

        yi = lax.bitcast_convert_type(d2, jnp.int32)
        y = lax.bitcast_convert_type(
            (yi >> 1) + jnp.int32(0x1FBD1DF5), jnp.float32)
        y = 0.5 * (y + d2 / y)
        y = 0.5 * (y + d2 / y)
        y = 0.5 * (y + d2 / y)
        ob[s] = y
        return carry

    lax.fori_loop(0, PV, pred_loop, 0, unroll=1)
    pltpu.sync_copy(ob, out_hbm.at[pl.ds(w * CHUNK, CHUNK)])


_sc_gather = pl.kernel(
    _gather_body,
    out_type=jax.ShapeDtypeStruct((NB * N,), jnp.float32),
    mesh=plsc.VectorSubcoreMesh(core_axis_name="c", subcore_axis_name="s"),
    compiler_params=pltpu.CompilerParams(needs_layout_passes=False),
    scratch_types=[
        pltpu.VMEM((CHUNK,), jnp.float32),
        pltpu.VMEM((CHUNK,), jnp.float32),
        pltpu.VMEM((CHUNK,), jnp.float32),
        pltpu.VMEM((N,), jnp.float32),
        pltpu.VMEM((N,), jnp.float32),
        pltpu.VMEM((N,), jnp.float32),
        pltpu.VMEM((CHUNK,), jnp.int32),
        pltpu.VMEM((CHUNK,), jnp.float32),
        pltpu.SemaphoreType.DMA,
    ],
)


@jax.jit
def kernel(pred, target):
    B, T, d, n = pred.shape
    pred2 = pred.reshape(NB, 3, N)
    target2 = target.reshape(NB, 3, N)
    idx = _tc_argmin(pred2, target2).reshape(NB * N)
    out = _sc_gather(pred2.reshape(-1), target2.reshape(-1), idx)
    return out.reshape(NB, N)


# speedup vs baseline: 1.9379x; 1.0001x over previous
import jax
import jax.numpy as jnp
from jax import lax
from jax.experimental import pallas as pl
from jax.experimental.pallas import tpu as pltpu
from jax.experimental.pallas import tpu_sc as plsc

L = 16
NW = 32
NB = 8
N = 2048
CHUNK = (NB * N) // NW
WPB = N // CHUNK
PV = CHUNK // L
PC = 1024


def _argmin_body(pred_ref, tgt_ref, idx_ref):
    ap = pred_ref[0]
    t = tgt_ref[0]
    rb = (t[0] * t[0] + t[1] * t[1]) + t[2] * t[2]
    m2 = lax.dot_general(
        ap.astype(jnp.bfloat16),
        (t * 2.0).astype(jnp.bfloat16),
        (((0,), (0,)), ((), ())), preferred_element_type=jnp.float32)
    d = rb[None, :] - m2
    idx_ref[0, 0] = jnp.argmin(d, axis=1).astype(jnp.int32)


_tc_argmin = pl.pallas_call(
    _argmin_body,
    grid=(NB, N // PC),
    in_specs=[
        pl.BlockSpec((1, 3, PC), lambda b, i: (b, 0, i)),
        pl.BlockSpec((1, 3, N), lambda b, i: (b, 0, 0)),
    ],
    out_specs=pl.BlockSpec((1, 1, PC), lambda b, i: (b * (N // PC) + i, 0, 0)),
    out_shape=jax.ShapeDtypeStruct((NB * (N // PC), 1, PC), jnp.int32),
)


def _gather_body(pred_hbm, target_hbm, idx_hbm, out_hbm,
                 px, py, pz, tx, ty, tz, ib, ob, dsem):
    w = lax.axis_index("s") * 2 + lax.axis_index("c")
    b = w // WPB
    off = (w % WPB) * CHUNK

    cps = [
        pltpu.async_copy(pred_hbm.at[pl.ds((b * 3 + 0) * N + off, CHUNK)],
                         px, dsem),
        pltpu.async_copy(pred_hbm.at[pl.ds((b * 3 + 1) * N + off, CHUNK)],
                         py, dsem),
        pltpu.async_copy(pred_hbm.at[pl.ds((b * 3 + 2) * N + off, CHUNK)],
                         pz, dsem),
        pltpu.async_copy(target_hbm.at[pl.ds((b * 3 + 0) * N, N)], tx, dsem),
        pltpu.async_copy(target_hbm.at[pl.ds((b * 3 + 1) * N, N)], ty, dsem),
        pltpu.async_copy(target_hbm.at[pl.ds((b * 3 + 2) * N, N)], tz, dsem),
        pltpu.async_copy(idx_hbm.at[pl.ds(w * CHUNK, CHUNK)], ib, dsem),
    ]
    for c in cps:
        c.wait()

    def pred_loop(ip, carry):
        s = pl.ds(ip * L, L)
        bj = ib[s]
        gx = plsc.load_gather(tx, [bj])
        gy = plsc.load_gather(ty, [bj])
        gz = plsc.load_gather(tz, [bj])
        dx = px[s] - gx
        dy = py[s] - gy
        dz = pz[s] - gz
        d2 = (dx * dx + dy * dy) + dz * dz

        yi = lax.bitcast_convert_type(d2, jnp.int32)
        y = lax.bitcast_convert_type(
            (yi >> 1) + jnp.int32(0x1FBD1DF5), jnp.float32)
        y = 0.5 * (y + d2 / y)
        y = 0.5 * (y + d2 / y)
        y = 0.5 * (y + d2 / y)
        ob[s] = y
        return carry

    lax.fori_loop(0, PV, pred_loop, 0, unroll=1)
    pltpu.sync_copy(ob, out_hbm.at[pl.ds(w * CHUNK, CHUNK)])


def _make_sc_gather():
    return pl.kernel(
        _gather_body,
        out_type=jax.ShapeDtypeStruct((NB * N,), jnp.float32),
        mesh=plsc.VectorSubcoreMesh(core_axis_name="c", subcore_axis_name="s"),
        compiler_params=pltpu.CompilerParams(needs_layout_passes=False),
        scratch_types=[
            pltpu.VMEM((CHUNK,), jnp.float32),
            pltpu.VMEM((CHUNK,), jnp.float32),
            pltpu.VMEM((CHUNK,), jnp.float32),
            pltpu.VMEM((N,), jnp.float32),
            pltpu.VMEM((N,), jnp.float32),
            pltpu.VMEM((N,), jnp.float32),
            pltpu.VMEM((CHUNK,), jnp.int32),
            pltpu.VMEM((CHUNK,), jnp.float32),
            pltpu.SemaphoreType.DMA,
        ],
    )


@jax.jit
def kernel(pred, target):
    B, T, d, n = pred.shape
    pred2 = pred.reshape(NB, 3, N)
    target2 = target.reshape(NB, 3, N)
    idx = _tc_argmin(pred2, target2).reshape(NB * N)
    out = _make_sc_gather()(pred2.reshape(-1), target2.reshape(-1), idx)
    return out.reshape(NB, N)
